# baseline (speedup 1.0000x reference)
import functools

import jax
import jax.numpy as jnp
from jax import lax
from jax.experimental import pallas as pl
from jax.experimental.pallas import tpu as pltpu
from jax.experimental.pallas import tpu_sc as plsc

_N = 10000
_E = 320000
_D = 128
_CHUNK = 80
_NC, _NS = 2, 16
_CPT = _E // _NS // _CHUNK
_HALF = _N // _NC
_ACC = _HALF + 8
_RPT = 312
_ZROWS = _ACC - (_NS - 1) * _RPT
_NBUF = 2
_R = 1000


def _remap(to_v, lo, c):
    for k in range(_CHUNK // 16):
        t = to_v[c, pl.ds(k * 16, 16)]
        loc = t - lo
        ok = (t >= lo) & (t < lo + _HALF)
        to_v[c, pl.ds(k * 16, 16)] = jnp.where(ok, loc, _HALF)


def _seg_body(*refs):
    (h_hbm, fro_hbm, to_hbm, z_hbm, out_hbm, fro_v, to_v, *rest) = refs
    rows = rest[:_NBUF]
    acc_sh = rest[_NBUF]
    ssem = rest[_NBUF + 1:_NBUF + 1 + _NBUF]

    cid = lax.axis_index("c")
    sid = lax.axis_index("s")
    lo = cid * _HALF
    r0 = sid * _RPT

    pltpu.sync_copy(z_hbm.at[pl.ds(0, _RPT)], acc_sh.at[pl.ds(r0, _RPT)])

    @pl.when(sid == _NS - 1)
    def _zero_rem():
        rr = _NS * _RPT
        pltpu.sync_copy(z_hbm.at[pl.ds(0, _ACC - rr)],
                        acc_sh.at[pl.ds(rr, _ACC - rr)])

    pltpu.sync_copy(fro_hbm.at[sid], fro_v)
    pltpu.sync_copy(to_hbm.at[sid], to_v)

    def remap_body(c, carry):
        _remap(to_v, lo, c)
        return carry

    lax.fori_loop(0, _CPT, remap_body, 0)
    plsc.subcore_barrier()

    def gather(c, b):
        pltpu.sync_copy(h_hbm.at[fro_v.at[c]], rows[b])

    def start_s(c, b):
        pltpu.async_copy(rows[b], acc_sh.at[to_v.at[c]], ssem[b], add=True)

    def wait_s(c, b):
        pltpu.make_async_copy(rows[b], acc_sh.at[to_v.at[c]], ssem[b]).wait()

    for c in range(2):
        gather(c, c)
        start_s(c, c)

    def body(g, carry):
        base = 2 * g
        for k in range(2):
            c = base + k
            wait_s(c - 2, k)
            gather(c, k)
            start_s(c, k)
        return carry

    lax.fori_loop(1, _CPT // 2, body, 0)
    for c in range(_CPT - 2, _CPT):
        wait_s(c, c % _NBUF)

    plsc.subcore_barrier()
    pltpu.sync_copy(acc_sh.at[pl.ds(r0, _RPT)],
                    out_hbm.at[pl.ds(cid * _HALF + r0, _RPT)])

    @pl.when(sid == _NS - 1)
    def _out_rem():
        rr = _NS * _RPT
        pltpu.sync_copy(acc_sh.at[pl.ds(rr, _HALF - rr)],
                        out_hbm.at[pl.ds(cid * _HALF + rr, _HALF - rr)])


_seg_kernel = pl.kernel(
    _seg_body,
    out_type=jax.ShapeDtypeStruct((_N, _D), jnp.float32),
    mesh=plsc.VectorSubcoreMesh(core_axis_name="c", subcore_axis_name="s"),
    scratch_types=(
        [pltpu.VMEM((_CPT, _CHUNK), jnp.int32),
         pltpu.VMEM((_CPT, _CHUNK), jnp.int32)]
        + [pltpu.VMEM((_CHUNK, _D), jnp.float32)] * _NBUF
        + [pltpu.VMEM_SHARED((_ACC, _D), jnp.float32)]
        + [pltpu.SemaphoreType.DMA] * _NBUF
    ),
    name="seg_sum",
)


def _cnt_body(to_hbm, z16_hbm, ones_hbm, cnt_hbm, to_v, ones_v, cnt_sh, csem):
    cid = lax.axis_index("c")
    sid = lax.axis_index("s")
    lo = cid * _HALF
    r0 = sid * _RPT

    pltpu.sync_copy(z16_hbm.at[pl.ds(0, _RPT)], cnt_sh.at[pl.ds(r0, _RPT)])

    @pl.when(sid == _NS - 1)
    def _zero_rem():
        rr = _NS * _RPT
        pltpu.sync_copy(z16_hbm.at[pl.ds(0, _ACC - rr)],
                        cnt_sh.at[pl.ds(rr, _ACC - rr)])

    pltpu.sync_copy(ones_hbm, ones_v)
    pltpu.sync_copy(to_hbm.at[sid], to_v)

    def remap_body(c, carry):
        _remap(to_v, lo, c)
        return carry

    lax.fori_loop(0, _CPT, remap_body, 0)
    plsc.subcore_barrier()

    for c in range(8):
        pltpu.async_copy(ones_v, cnt_sh.at[to_v.at[c]], csem, add=True)

    def fire(c, carry):
        pltpu.make_async_copy(ones_v, cnt_sh.at[to_v.at[c - 8]], csem).wait()
        pltpu.async_copy(ones_v, cnt_sh.at[to_v.at[c]], csem, add=True)
        return carry

    lax.fori_loop(8, _CPT, fire, 0)

    def drain(c, carry):
        pltpu.make_async_copy(ones_v, cnt_sh.at[to_v.at[c]], csem).wait()
        return carry

    lax.fori_loop(_CPT - 8, _CPT, drain, 0)

    plsc.subcore_barrier()
    pltpu.sync_copy(cnt_sh.at[pl.ds(r0, _RPT)],
                    cnt_hbm.at[pl.ds(cid * _HALF + r0, _RPT)])

    @pl.when(sid == _NS - 1)
    def _out_rem():
        rr = _NS * _RPT
        pltpu.sync_copy(cnt_sh.at[pl.ds(rr, _HALF - rr)],
                        cnt_hbm.at[pl.ds(cid * _HALF + rr, _HALF - rr)])


_cnt_kernel = pl.kernel(
    _cnt_body,
    out_type=jax.ShapeDtypeStruct((_N, 16), jnp.float32),
    mesh=plsc.VectorSubcoreMesh(core_axis_name="c", subcore_axis_name="s"),
    scratch_types=[
        pltpu.VMEM((_CPT, _CHUNK), jnp.int32),
        pltpu.VMEM((_CHUNK, 16), jnp.float32),
        pltpu.VMEM_SHARED((_ACC, 16), jnp.float32),
        pltpu.SemaphoreType.DMA,
    ],
    name="seg_counts",
)


def _init_body(x_ref, w_ref, b_ref, o_ref):
    o_ref[...] = jax.nn.relu(
        jnp.dot(x_ref[...], w_ref[...], preferred_element_type=jnp.float32)
        + b_ref[...])


def _combine_body(h_ref, p_ref, c_ref, ws_ref, wm_ref, b_ref, o_ref):
    agg = p_ref[...] / jnp.maximum(c_ref[:, 0:1], 1.0)
    o_ref[...] = jax.nn.relu(
        jnp.dot(h_ref[...], ws_ref[...], preferred_element_type=jnp.float32)
        + jnp.dot(agg, wm_ref[...], preferred_element_type=jnp.float32)
        + b_ref[...])


def _final_body(h_ref, p_ref, c_ref, ws_ref, wm_ref, b_ref, wo_ref, bo_ref,
                o_ref):
    agg = p_ref[...] / jnp.maximum(c_ref[:, 0:1], 1.0)
    h1 = jax.nn.relu(
        jnp.dot(h_ref[...], ws_ref[...], preferred_element_type=jnp.float32)
        + jnp.dot(agg, wm_ref[...], preferred_element_type=jnp.float32)
        + b_ref[...])
    o_ref[...] = (jnp.dot(h1, wo_ref[...], preferred_element_type=jnp.float32)
                  + bo_ref[...])


_row_spec = pl.BlockSpec((_R, _D), lambda i: (i, 0))
_c_spec = pl.BlockSpec((_R, 16), lambda i: (i, 0))
_w_spec = pl.BlockSpec((_D, _D), lambda i: (0, 0))
_b_spec = pl.BlockSpec((1, _D), lambda i: (0, 0))
_out_struct = jax.ShapeDtypeStruct((_N, _D), jnp.float32)

_init_mm = pl.pallas_call(
    _init_body,
    grid=(_N // _R,),
    in_specs=[_row_spec, _w_spec, _b_spec],
    out_specs=_row_spec,
    out_shape=_out_struct,
)

_combine_mm = pl.pallas_call(
    _combine_body,
    grid=(_N // _R,),
    in_specs=[_row_spec, _row_spec, _c_spec, _w_spec, _w_spec, _b_spec],
    out_specs=_row_spec,
    out_shape=_out_struct,
)

_final_mm = pl.pallas_call(
    _final_body,
    grid=(_N // _R,),
    in_specs=[_row_spec, _row_spec, _c_spec, _w_spec, _w_spec, _b_spec,
              _w_spec, _b_spec],
    out_specs=_row_spec,
    out_shape=_out_struct,
)


def kernel(nodes_feats, adj_list, W_init, b_init, W_self, W_msg, b_layers,
           W_out, b_out):
    fro = adj_list[:, 0].reshape(_NS, _CPT, _CHUNK)
    to = adj_list[:, 1].reshape(_NS, _CPT, _CHUNK)
    z128 = jnp.zeros((_ZROWS, _D), jnp.float32)
    z16 = jnp.zeros((_ZROWS, 16), jnp.float32)
    ones16 = jnp.ones((_CHUNK, 16), jnp.float32)

    cnts = _cnt_kernel(to, z16, ones16)
    h = _init_mm(nodes_feats, W_init, b_init.reshape(1, _D))
    parts = _seg_kernel(h, fro, to, z128)
    h = _combine_mm(h, parts, cnts, W_self[0], W_msg[0],
                    b_layers[0].reshape(1, _D))
    parts = _seg_kernel(h, fro, to, z128)
    return _final_mm(h, parts, cnts, W_self[1], W_msg[1],
                     b_layers[1].reshape(1, _D), W_out, b_out.reshape(1, _D))


# speedup vs baseline: 1.1103x; 1.1103x over previous
import jax
import jax.numpy as jnp
from jax import lax
from jax.experimental import pallas as pl
from jax.experimental.pallas import tpu as pltpu
from jax.experimental.pallas import tpu_sc as plsc

_N = 10000
_E = 320000
_D = 128
_CHUNK = 80
_NC, _NS = 2, 16
_CPT = _E // _NS // _CHUNK
_HALF = _N // _NC
_ACC = _HALF + 8
_RPT = 312
_ZROWS = _ACC - (_NS - 1) * _RPT
_R = 1000


def _remap(to_v, lo, c):
    for k in range(_CHUNK // 16):
        t = to_v[c, pl.ds(k * 16, 16)]
        loc = t - lo
        ok = (t >= lo) & (t < lo + _HALF)
        to_v[c, pl.ds(k * 16, 16)] = jnp.where(ok, loc, _HALF)


def _seg_body(h_hbm, fro_hbm, to_hbm, z_hbm, out_hbm,
              fro_v, to_v, rows0, rows1, acc_sh, sem0, sem1):
    cid = lax.axis_index("c")
    sid = lax.axis_index("s")
    lo = cid * _HALF
    r0 = sid * _RPT

    pltpu.sync_copy(z_hbm.at[pl.ds(0, _RPT)], acc_sh.at[pl.ds(r0, _RPT)])

    @pl.when(sid == _NS - 1)
    def _zero_rem():
        rr = _NS * _RPT
        pltpu.sync_copy(z_hbm.at[pl.ds(0, _ACC - rr)],
                        acc_sh.at[pl.ds(rr, _ACC - rr)])

    pltpu.sync_copy(fro_hbm.at[sid], fro_v)
    pltpu.sync_copy(to_hbm.at[sid], to_v)

    def remap_body(c, carry):
        _remap(to_v, lo, c)
        return carry

    lax.fori_loop(0, _CPT, remap_body, 0)
    plsc.subcore_barrier()

    def start(c, rows, sem):
        pltpu.async_copy(h_hbm.at[fro_v.at[c]], rows, sem)

    def wait(c, rows, sem):
        pltpu.make_async_copy(h_hbm.at[fro_v.at[c]], rows, sem).wait()

    def scat(c, rows):
        pltpu.sync_copy(rows, acc_sh.at[to_v.at[c]], add=True)

    start(0, rows0, sem0)

    def body(g, carry):
        c = 2 * g
        start(c + 1, rows1, sem1)
        wait(c, rows0, sem0)
        scat(c, rows0)
        start(c + 2, rows0, sem0)
        wait(c + 1, rows1, sem1)
        scat(c + 1, rows1)
        return carry

    lax.fori_loop(0, _CPT // 2 - 1, body, 0)
    c = _CPT - 2
    start(c + 1, rows1, sem1)
    wait(c, rows0, sem0)
    scat(c, rows0)
    wait(c + 1, rows1, sem1)
    scat(c + 1, rows1)

    plsc.subcore_barrier()
    pltpu.sync_copy(acc_sh.at[pl.ds(r0, _RPT)],
                    out_hbm.at[pl.ds(cid * _HALF + r0, _RPT)])

    @pl.when(sid == _NS - 1)
    def _out_rem():
        rr = _NS * _RPT
        pltpu.sync_copy(acc_sh.at[pl.ds(rr, _HALF - rr)],
                        out_hbm.at[pl.ds(cid * _HALF + rr, _HALF - rr)])


_seg_kernel = pl.kernel(
    _seg_body,
    out_type=jax.ShapeDtypeStruct((_N, _D), jnp.float32),
    mesh=plsc.VectorSubcoreMesh(core_axis_name="c", subcore_axis_name="s"),
    scratch_types=[
        pltpu.VMEM((_CPT, _CHUNK), jnp.int32),
        pltpu.VMEM((_CPT, _CHUNK), jnp.int32),
        pltpu.VMEM((_CHUNK, _D), jnp.float32),
        pltpu.VMEM((_CHUNK, _D), jnp.float32),
        pltpu.VMEM_SHARED((_ACC, _D), jnp.float32),
        pltpu.SemaphoreType.DMA,
        pltpu.SemaphoreType.DMA,
    ],
    name="seg_sum",
)


def _cnt_body(to_hbm, z16_hbm, ones_hbm, cnt_hbm, to_v, ones_v, cnt_sh, csem):
    cid = lax.axis_index("c")
    sid = lax.axis_index("s")
    lo = cid * _HALF
    r0 = sid * _RPT

    pltpu.sync_copy(z16_hbm.at[pl.ds(0, _RPT)], cnt_sh.at[pl.ds(r0, _RPT)])

    @pl.when(sid == _NS - 1)
    def _zero_rem():
        rr = _NS * _RPT
        pltpu.sync_copy(z16_hbm.at[pl.ds(0, _ACC - rr)],
                        cnt_sh.at[pl.ds(rr, _ACC - rr)])

    pltpu.sync_copy(ones_hbm, ones_v)
    pltpu.sync_copy(to_hbm.at[sid], to_v)

    def remap_body(c, carry):
        _remap(to_v, lo, c)
        return carry

    lax.fori_loop(0, _CPT, remap_body, 0)
    plsc.subcore_barrier()

    for c in range(8):
        pltpu.async_copy(ones_v, cnt_sh.at[to_v.at[c]], csem, add=True)

    def fire(c, carry):
        pltpu.make_async_copy(ones_v, cnt_sh.at[to_v.at[c - 8]], csem).wait()
        pltpu.async_copy(ones_v, cnt_sh.at[to_v.at[c]], csem, add=True)
        return carry

    lax.fori_loop(8, _CPT, fire, 0)

    def drain(c, carry):
        pltpu.make_async_copy(ones_v, cnt_sh.at[to_v.at[c]], csem).wait()
        return carry

    lax.fori_loop(_CPT - 8, _CPT, drain, 0)

    plsc.subcore_barrier()
    pltpu.sync_copy(cnt_sh.at[pl.ds(r0, _RPT)],
                    cnt_hbm.at[pl.ds(cid * _HALF + r0, _RPT)])

    @pl.when(sid == _NS - 1)
    def _out_rem():
        rr = _NS * _RPT
        pltpu.sync_copy(cnt_sh.at[pl.ds(rr, _HALF - rr)],
                        cnt_hbm.at[pl.ds(cid * _HALF + rr, _HALF - rr)])


_cnt_kernel = pl.kernel(
    _cnt_body,
    out_type=jax.ShapeDtypeStruct((_N, 16), jnp.float32),
    mesh=plsc.VectorSubcoreMesh(core_axis_name="c", subcore_axis_name="s"),
    scratch_types=[
        pltpu.VMEM((_CPT, _CHUNK), jnp.int32),
        pltpu.VMEM((_CHUNK, 16), jnp.float32),
        pltpu.VMEM_SHARED((_ACC, 16), jnp.float32),
        pltpu.SemaphoreType.DMA,
    ],
    name="seg_counts",
)


def _init_body(x_ref, w_ref, b_ref, o_ref):
    o_ref[...] = jax.nn.relu(
        jnp.dot(x_ref[...], w_ref[...], preferred_element_type=jnp.float32)
        + b_ref[...])


def _combine_body(h_ref, p_ref, c_ref, ws_ref, wm_ref, b_ref, o_ref):
    agg = p_ref[...] / jnp.maximum(c_ref[:, 0:1], 1.0)
    o_ref[...] = jax.nn.relu(
        jnp.dot(h_ref[...], ws_ref[...], preferred_element_type=jnp.float32)
        + jnp.dot(agg, wm_ref[...], preferred_element_type=jnp.float32)
        + b_ref[...])


def _final_body(h_ref, p_ref, c_ref, ws_ref, wm_ref, b_ref, wo_ref, bo_ref,
                o_ref):
    agg = p_ref[...] / jnp.maximum(c_ref[:, 0:1], 1.0)
    h1 = jax.nn.relu(
        jnp.dot(h_ref[...], ws_ref[...], preferred_element_type=jnp.float32)
        + jnp.dot(agg, wm_ref[...], preferred_element_type=jnp.float32)
        + b_ref[...])
    o_ref[...] = (jnp.dot(h1, wo_ref[...], preferred_element_type=jnp.float32)
                  + bo_ref[...])


_row_spec = pl.BlockSpec((_R, _D), lambda i: (i, 0))
_c_spec = pl.BlockSpec((_R, 16), lambda i: (i, 0))
_w_spec = pl.BlockSpec((_D, _D), lambda i: (0, 0))
_b_spec = pl.BlockSpec((1, _D), lambda i: (0, 0))
_out_struct = jax.ShapeDtypeStruct((_N, _D), jnp.float32)

_init_mm = pl.pallas_call(
    _init_body,
    grid=(_N // _R,),
    in_specs=[_row_spec, _w_spec, _b_spec],
    out_specs=_row_spec,
    out_shape=_out_struct,
)

_combine_mm = pl.pallas_call(
    _combine_body,
    grid=(_N // _R,),
    in_specs=[_row_spec, _row_spec, _c_spec, _w_spec, _w_spec, _b_spec],
    out_specs=_row_spec,
    out_shape=_out_struct,
)

_final_mm = pl.pallas_call(
    _final_body,
    grid=(_N // _R,),
    in_specs=[_row_spec, _row_spec, _c_spec, _w_spec, _w_spec, _b_spec,
              _w_spec, _b_spec],
    out_specs=_row_spec,
    out_shape=_out_struct,
)


def kernel(nodes_feats, adj_list, W_init, b_init, W_self, W_msg, b_layers,
           W_out, b_out):
    fro = adj_list[:, 0].reshape(_NS, _CPT, _CHUNK)
    to = adj_list[:, 1].reshape(_NS, _CPT, _CHUNK)
    z128 = jnp.zeros((_ZROWS, _D), jnp.float32)
    z16 = jnp.zeros((_ZROWS, 16), jnp.float32)
    ones16 = jnp.ones((_CHUNK, 16), jnp.float32)

    cnts = _cnt_kernel(to, z16, ones16)
    h = _init_mm(nodes_feats, W_init, b_init.reshape(1, _D))
    parts = _seg_kernel(h, fro, to, z128)
    h = _combine_mm(h, parts, cnts, W_self[0], W_msg[0],
                    b_layers[0].reshape(1, _D))
    parts = _seg_kernel(h, fro, to, z128)
    return _final_mm(h, parts, cnts, W_self[1], W_msg[1],
                     b_layers[1].reshape(1, _D), W_out, b_out.reshape(1, _D))
